# baseline (speedup 1.0000x reference)
import functools

import jax
import jax.numpy as jnp
from jax import lax
from jax.experimental import pallas as pl
from jax.experimental.pallas import tpu as pltpu
from jax.experimental.pallas import tpu_sc as plsc

VOCAB = 1000000
EMBED_DIM = 64
PAD_DIM = 128
BATCH = 4096
HIST = 50

N = BATCH * HIST
NC = 2
NS = 16
NW = NC * NS
PER_W = N // NW
CHUNK = 400
NCHUNK = PER_W // CHUNK


def _body(x1_hbm, x2_hbm, table_hbm, out1_hbm, out2_hbm,
          idx0, idx1, rows0, rows1, sem0, sem1):
    wid = lax.axis_index("s") * NC + lax.axis_index("c")
    base = wid * PER_W

    idx_bufs = (idx0, idx1)
    row_bufs = (rows0, rows1)
    sems = (sem0, sem1)

    def start(b, off):
        pltpu.sync_copy(start.src.at[pl.ds(off, CHUNK)], idx_bufs[b])
        pltpu.async_copy(table_hbm.at[idx_bufs[b]], row_bufs[b], sems[b])

    def finish(b, off):
        pltpu.make_async_copy(table_hbm.at[idx_bufs[b]], row_bufs[b],
                              sems[b]).wait()
        pltpu.sync_copy(row_bufs[b], finish.dst.at[pl.ds(off, CHUNK)])

    for src, dst in ((x1_hbm, out1_hbm), (x2_hbm, out2_hbm)):
        start.src, finish.dst = src, dst
        start(0, base)
        start(1, base + CHUNK)

        def steady(m, _):
            for b in range(2):
                off = base + (2 * m + b) * CHUNK
                finish(b, off)
                start(b, off + 2 * CHUNK)
            return 0

        lax.fori_loop(0, NCHUNK // 2 - 1, steady, 0)
        for k in (NCHUNK - 2, NCHUNK - 1):
            finish(k & 1, base + k * CHUNK)


_sc_kernel = functools.partial(
    pl.kernel,
    out_type=(jax.ShapeDtypeStruct((N, PAD_DIM), jnp.float32),
              jax.ShapeDtypeStruct((N, PAD_DIM), jnp.float32)),
    mesh=plsc.VectorSubcoreMesh(core_axis_name="c", subcore_axis_name="s"),
    scratch_types=[
        pltpu.VMEM((CHUNK,), jnp.int32),
        pltpu.VMEM((CHUNK,), jnp.int32),
        pltpu.VMEM((CHUNK, PAD_DIM), jnp.float32),
        pltpu.VMEM((CHUNK, PAD_DIM), jnp.float32),
        pltpu.SemaphoreType.DMA,
        pltpu.SemaphoreType.DMA,
    ],
    compiler_params=pltpu.CompilerParams(use_tc_tiling_on_sc=False),
)(_body)


def kernel(x1, x2, table):
    tp = jnp.pad(table, ((0, 0), (0, PAD_DIM - EMBED_DIM)))
    f1 = x1.reshape(-1).astype(jnp.int32)
    f2 = x2.reshape(-1).astype(jnp.int32)
    o1, o2 = _sc_kernel(f1, f2, tp)
    return (o1[:, :EMBED_DIM].reshape(BATCH, HIST, EMBED_DIM),
            o2[:, :EMBED_DIM].reshape(BATCH, HIST, EMBED_DIM))
